# baseline (speedup 1.0000x reference)
import functools

import jax
import jax.numpy as jnp
from jax.experimental import pallas as pl
from jax.experimental.pallas import tpu as pltpu

HI = jax.lax.Precision.HIGHEST


def _dot_t(a, b, precision=HI):
    return jax.lax.dot_general(a, b, (((1,), (1,)), ((), ())),
                               precision=precision,
                               preferred_element_type=jnp.float32)


def _layernorm(x, g, b, eps=1e-5):
    m = x.mean(-1, keepdims=True)
    v = ((x - m) ** 2).mean(-1, keepdims=True)
    return (x - m) * jax.lax.rsqrt(v + eps) * g + b


def _kv_kernel(x_ref, g_ref, b_ref, wk_ref, wv_ref, k_ref, v_ref):
    xn = _layernorm(x_ref[...], g_ref[...], b_ref[...])
    k_ref[...] = _dot_t(xn, wk_ref[...])
    v_ref[...] = _dot_t(xn, wv_ref[...])


def _attn_kernel(slots_ref, k_ref, v_ref, wq_ref, g_ref, b_ref,
                 attn_ref, upd_ref, *, scale):
    sn = _layernorm(slots_ref[...], g_ref[...], b_ref[...])
    q = _dot_t(sn, wq_ref[...])
    logits = _dot_t(q, k_ref[...]) * scale
    m = jnp.max(logits, axis=0, keepdims=True)
    e = jnp.exp(logits - m)
    attn = e / jnp.sum(e, axis=0, keepdims=True)
    attn_ref[...] = attn
    w = attn / (jnp.sum(attn, axis=1, keepdims=True) + 1e-8)
    upd_ref[...] = jax.lax.dot_general(
        w, v_ref[...], (((1,), (0,)), ((), ())),
        precision=HI, preferred_element_type=jnp.float32)


def _gru_mlp_kernel(u_ref, h_ref, wih_ref, whh_ref, bih_ref, bhh_ref,
                    g_ref, b_ref, w1_ref, b1_ref, w2_ref, b2_ref, out_ref):
    u = u_ref[...]
    h = h_ref[...]
    D = u.shape[-1]
    gi = _dot_t(u, wih_ref[...]) + bih_ref[...]
    gh = _dot_t(h, whh_ref[...]) + bhh_ref[...]
    r = jax.nn.sigmoid(gi[:, :D] + gh[:, :D])
    z = jax.nn.sigmoid(gi[:, D:2 * D] + gh[:, D:2 * D])
    n = jnp.tanh(gi[:, 2 * D:] + r * gh[:, 2 * D:])
    s = (1.0 - z) * n + z * h
    hh = _layernorm(s, g_ref[...], b_ref[...])
    h1 = jnp.maximum(_dot_t(hh, w1_ref[...]) + b1_ref[...], 0.0)
    out_ref[...] = s + _dot_t(h1, w2_ref[...]) + b2_ref[...]


def _sel_quality_kernel(slots_ref, attn_ref, sw1_ref, sb1_ref, sw2_ref,
                        sb2_ref, qw1_ref, qb1_ref, qw2_ref, qb2_ref,
                        mask_ref, qual_ref, *, B, K):
    s2 = slots_ref[...]
    sh = jnp.maximum(_dot_t(s2, sw1_ref[...]) + sb1_ref[...], 0.0)
    sel_logits = (jnp.sum(sh * sw2_ref[...], axis=1, keepdims=True)
                  + sb2_ref[0, 0])
    sel_probs = jax.nn.sigmoid(sel_logits)
    mask = jnp.where(sel_probs > 0.5, 1.0, 0.0)

    qh = jnp.maximum(_dot_t(s2, qw1_ref[...]) + qb1_ref[...], 0.0)
    learned_q = jax.nn.sigmoid(
        jnp.sum(qh * qw2_ref[...], axis=1, keepdims=True) + qb2_ref[0, 0])

    row_iota = jax.lax.broadcasted_iota(jnp.int32, (K, 1), 0)
    min_sum = None
    for b in range(B):
        lo = b * K
        lg = sel_logits[lo:lo + K, :]
        msum = jnp.sum(mask[lo:lo + K, :])
        min_sum = msum if min_sum is None else jnp.minimum(min_sum, msum)
        mx = jnp.max(lg)
        idx = jnp.min(jnp.where(lg == mx, row_iota, K))
        fb = jnp.where(row_iota == idx, 1.0, 0.0)
        a = attn_ref[b]
        amax = jnp.max(a, axis=0, keepdims=True)
        kio = jax.lax.broadcasted_iota(jnp.int32, a.shape, 0)
        widx = jnp.min(jnp.where(a == amax, kio, K),
                       axis=0, keepdims=True)
        onehot = jnp.where(kio == widx, 1.0, 0.0)
        winning = jnp.sum(a * onehot, axis=1, keepdims=True)
        total = jnp.sum(a, axis=1, keepdims=True)
        aq = winning / (total + 1e-8)
        qual_ref[lo:lo + K, :] = 0.4 * aq + 0.4 * learned_q[lo:lo + K, :]
        mask_ref[lo:lo + K, :] = fb

    need_fb = min_sum < 1.0
    for b in range(B):
        lo = b * K
        mask_ref[lo:lo + K, :] = jnp.where(
            need_fb, mask_ref[lo:lo + K, :], mask[lo:lo + K, :])


def kernel(inputs, params):
    B, N, D = inputs.shape
    K = params['slot_inits'].shape[0]
    scale = D ** -0.5
    f32 = jnp.float32

    def row(p):
        return p.reshape(1, -1)

    TM = 256
    x2 = inputs.reshape(B * N, D)
    kv = pl.pallas_call(
        _kv_kernel,
        grid=(B * N // TM,),
        in_specs=[
            pl.BlockSpec((TM, D), lambda i: (i, 0)),
            pl.BlockSpec((1, D), lambda i: (0, 0)),
            pl.BlockSpec((1, D), lambda i: (0, 0)),
            pl.BlockSpec((D, D), lambda i: (0, 0)),
            pl.BlockSpec((D, D), lambda i: (0, 0)),
        ],
        out_specs=[
            pl.BlockSpec((TM, D), lambda i: (i, 0)),
            pl.BlockSpec((TM, D), lambda i: (i, 0)),
        ],
        out_shape=[
            jax.ShapeDtypeStruct((B * N, D), f32),
            jax.ShapeDtypeStruct((B * N, D), f32),
        ],
    )
    k2, v2 = kv(x2, row(params['in_ln_g']), row(params['in_ln_b']),
                params['Wk'], params['Wv'])
    k3 = k2.reshape(B, N, D)
    v3 = v2.reshape(B, N, D)

    attn_call = pl.pallas_call(
        functools.partial(_attn_kernel, scale=scale),
        grid=(B,),
        in_specs=[
            pl.BlockSpec((None, K, D), lambda b: (b, 0, 0)),
            pl.BlockSpec((None, N, D), lambda b: (b, 0, 0)),
            pl.BlockSpec((None, N, D), lambda b: (b, 0, 0)),
            pl.BlockSpec((D, D), lambda b: (0, 0)),
            pl.BlockSpec((1, D), lambda b: (0, 0)),
            pl.BlockSpec((1, D), lambda b: (0, 0)),
        ],
        out_specs=[
            pl.BlockSpec((None, K, N), lambda b: (b, 0, 0)),
            pl.BlockSpec((None, K, D), lambda b: (b, 0, 0)),
        ],
        out_shape=[
            jax.ShapeDtypeStruct((B, K, N), f32),
            jax.ShapeDtypeStruct((B, K, D), f32),
        ],
    )

    full = lambda shape: pl.BlockSpec(shape, lambda: tuple(0 for _ in shape))
    gru_call = pl.pallas_call(
        _gru_mlp_kernel,
        in_specs=[
            full((B * K, D)), full((B * K, D)),
            full((3 * D, D)), full((3 * D, D)),
            full((1, 3 * D)), full((1, 3 * D)),
            full((1, D)), full((1, D)),
            full((2 * D, D)), full((1, 2 * D)),
            full((D, 2 * D)), full((1, D)),
        ],
        out_specs=full((B * K, D)),
        out_shape=jax.ShapeDtypeStruct((B * K, D), f32),
    )

    slots = jnp.broadcast_to(params['slot_inits'][None], (B, K, D)).astype(f32)
    attn = None
    for _ in range(3):
        attn, upd = attn_call(slots, k3, v3, params['Wq'],
                              row(params['slot_ln_g']), row(params['slot_ln_b']))
        s64 = gru_call(upd.reshape(B * K, D), slots.reshape(B * K, D),
                       params['W_ih'], params['W_hh'],
                       row(params['b_ih']), row(params['b_hh']),
                       row(params['mlp_ln_g']), row(params['mlp_ln_b']),
                       params['mlp_W1'], row(params['mlp_b1']),
                       params['mlp_W2'], row(params['mlp_b2']))
        slots = s64.reshape(B, K, D)

    sel_call = pl.pallas_call(
        functools.partial(_sel_quality_kernel, B=B, K=K),
        in_specs=[
            full((B * K, D)), full((B, K, N)),
            full((D // 2, D)), full((1, D // 2)),
            full((1, D // 2)), full((1, 1)),
            full((D // 4, D)), full((1, D // 4)),
            full((1, D // 4)), full((1, 1)),
        ],
        out_specs=[full((B * K, 1)), full((B * K, 1))],
        out_shape=[
            jax.ShapeDtypeStruct((B * K, 1), f32),
            jax.ShapeDtypeStruct((B * K, 1), f32),
        ],
    )
    mask64, qual64 = sel_call(
        slots.reshape(B * K, D), attn,
        params['sel_W1'], row(params['sel_b1']),
        params['sel_W2'], row(params['sel_b2']),
        params['q_W1'], row(params['q_b1']),
        params['q_W2'], row(params['q_b2']))

    return slots, mask64.reshape(B, K), qual64.reshape(B, K), attn


# speedup vs baseline: 3.7900x; 3.7900x over previous
import functools

import jax
import jax.numpy as jnp
from jax.experimental import pallas as pl
from jax.experimental.pallas import tpu as pltpu

HI = jax.lax.Precision.HIGHEST
BF = jnp.bfloat16


def _dot_t(a, b, precision=HI):
    return jax.lax.dot_general(a, b, (((1,), (1,)), ((), ())),
                               precision=precision,
                               preferred_element_type=jnp.float32)


def _bdot_t(a, b):
    return jax.lax.dot_general(a.astype(BF), b, (((1,), (1,)), ((), ())),
                               preferred_element_type=jnp.float32)


def _layernorm(x, g, b, eps=1e-5):
    m = x.mean(-1, keepdims=True)
    v = ((x - m) ** 2).mean(-1, keepdims=True)
    return (x - m) * jax.lax.rsqrt(v + eps) * g + b


def _kv_kernel(x_ref, g_ref, b_ref, wk_ref, wv_ref, k_ref, v_ref):
    xn = _layernorm(x_ref[...], g_ref[...], b_ref[...]).astype(BF)
    k_ref[...] = _bdot_t(xn, wk_ref[...]).astype(BF)
    v_ref[...] = _bdot_t(xn, wv_ref[...]).astype(BF)


def _iters_kernel(si_ref, k_ref, v_ref, wq_ref, sg_ref, sb_ref,
                  wih_ref, whh_ref, bih_ref, bhh_ref,
                  mg_ref, mb_ref, w1_ref, b1_ref, w2_ref, b2_ref,
                  slots_ref, attn_ref, *, B, K, N, D, scale, iters):
    si = si_ref[...]
    h = jnp.concatenate([si] * B, axis=0)
    for it in range(iters):
        sn = _layernorm(h, sg_ref[...], sb_ref[...])
        q = _bdot_t(sn, wq_ref[...])
        upds = []
        for b in range(B):
            qb = q[b * K:(b + 1) * K, :].astype(BF)
            logits = jax.lax.dot_general(
                qb, k_ref[b], (((1,), (1,)), ((), ())),
                preferred_element_type=jnp.float32) * scale
            m = jnp.max(logits, axis=0, keepdims=True)
            e = jnp.exp(logits - m)
            attn_b = e / jnp.sum(e, axis=0, keepdims=True)
            attn_ref[b] = attn_b
            w = attn_b / (jnp.sum(attn_b, axis=1, keepdims=True) + 1e-8)
            upds.append(jax.lax.dot_general(
                w.astype(BF), v_ref[b], (((1,), (0,)), ((), ())),
                preferred_element_type=jnp.float32))
        u = jnp.concatenate(upds, axis=0)
        gi = _bdot_t(u, wih_ref[...]) + bih_ref[...]
        gh = _bdot_t(h, whh_ref[...]) + bhh_ref[...]
        r = jax.nn.sigmoid(gi[:, :D] + gh[:, :D])
        z = jax.nn.sigmoid(gi[:, D:2 * D] + gh[:, D:2 * D])
        n = jnp.tanh(gi[:, 2 * D:] + r * gh[:, 2 * D:])
        s = (1.0 - z) * n + z * h
        hm = _layernorm(s, mg_ref[...], mb_ref[...])
        h1 = jnp.maximum(_bdot_t(hm, w1_ref[...]) + b1_ref[...], 0.0)
        h = s + _bdot_t(h1, w2_ref[...]) + b2_ref[...]
    slots_ref[...] = h


def _sel_quality_kernel(slots_ref, attn_ref, sw1_ref, sb1_ref, sw2_ref,
                        sb2_ref, qw1_ref, qb1_ref, qw2_ref, qb2_ref,
                        mask_ref, qual_ref, *, B, K):
    s2 = slots_ref[...]
    sh = jnp.maximum(_dot_t(s2, sw1_ref[...]) + sb1_ref[...], 0.0)
    sel_logits = (jnp.sum(sh * sw2_ref[...], axis=1, keepdims=True)
                  + sb2_ref[0, 0])
    sel_probs = jax.nn.sigmoid(sel_logits)
    mask = jnp.where(sel_probs > 0.5, 1.0, 0.0)

    qh = jnp.maximum(_dot_t(s2, qw1_ref[...]) + qb1_ref[...], 0.0)
    learned_q = jax.nn.sigmoid(
        jnp.sum(qh * qw2_ref[...], axis=1, keepdims=True) + qb2_ref[0, 0])

    row_iota = jax.lax.broadcasted_iota(jnp.int32, (K, 1), 0)
    min_sum = None
    for b in range(B):
        lo = b * K
        lg = sel_logits[lo:lo + K, :]
        msum = jnp.sum(mask[lo:lo + K, :])
        min_sum = msum if min_sum is None else jnp.minimum(min_sum, msum)
        mx = jnp.max(lg)
        idx = jnp.min(jnp.where(lg == mx, row_iota, K))
        fb = jnp.where(row_iota == idx, 1.0, 0.0)
        a = attn_ref[b]
        amax = jnp.max(a, axis=0, keepdims=True)
        kio = jax.lax.broadcasted_iota(jnp.int32, a.shape, 0)
        widx = jnp.min(jnp.where(a == amax, kio, K),
                       axis=0, keepdims=True)
        onehot = jnp.where(kio == widx, 1.0, 0.0)
        winning = jnp.sum(a * onehot, axis=1, keepdims=True)
        total = jnp.sum(a, axis=1, keepdims=True)
        aq = winning / (total + 1e-8)
        qual_ref[lo:lo + K, :] = 0.4 * aq + 0.4 * learned_q[lo:lo + K, :]
        mask_ref[lo:lo + K, :] = fb

    need_fb = min_sum < 1.0
    for b in range(B):
        lo = b * K
        mask_ref[lo:lo + K, :] = jnp.where(
            need_fb, mask_ref[lo:lo + K, :], mask[lo:lo + K, :])


def kernel(inputs, params):
    B, N, D = inputs.shape
    K = params['slot_inits'].shape[0]
    scale = D ** -0.5
    f32 = jnp.float32

    def row(p):
        return p.reshape(1, -1)

    TM = 512
    x2 = inputs.reshape(B * N, D)
    kv = pl.pallas_call(
        _kv_kernel,
        grid=(B * N // TM,),
        in_specs=[
            pl.BlockSpec((TM, D), lambda i: (i, 0)),
            pl.BlockSpec((1, D), lambda i: (0, 0)),
            pl.BlockSpec((1, D), lambda i: (0, 0)),
            pl.BlockSpec((D, D), lambda i: (0, 0)),
            pl.BlockSpec((D, D), lambda i: (0, 0)),
        ],
        out_specs=[
            pl.BlockSpec((TM, D), lambda i: (i, 0)),
            pl.BlockSpec((TM, D), lambda i: (i, 0)),
        ],
        out_shape=[
            jax.ShapeDtypeStruct((B * N, D), BF),
            jax.ShapeDtypeStruct((B * N, D), BF),
        ],
    )
    k2, v2 = kv(x2, row(params['in_ln_g']), row(params['in_ln_b']),
                params['Wk'].astype(BF), params['Wv'].astype(BF))
    k3 = k2.reshape(B, N, D)
    v3 = v2.reshape(B, N, D)

    full = lambda shape: pl.BlockSpec(shape, lambda: tuple(0 for _ in shape))
    iters_call = pl.pallas_call(
        functools.partial(_iters_kernel, B=B, K=K, N=N, D=D,
                          scale=scale, iters=3),
        in_specs=[
            full((K, D)), full((B, N, D)), full((B, N, D)),
            full((D, D)), full((1, D)), full((1, D)),
            full((3 * D, D)), full((3 * D, D)),
            full((1, 3 * D)), full((1, 3 * D)),
            full((1, D)), full((1, D)),
            full((2 * D, D)), full((1, 2 * D)),
            full((D, 2 * D)), full((1, D)),
        ],
        out_specs=[full((B * K, D)), full((B, K, N))],
        out_shape=[
            jax.ShapeDtypeStruct((B * K, D), f32),
            jax.ShapeDtypeStruct((B, K, N), f32),
        ],
        compiler_params=pltpu.CompilerParams(
            vmem_limit_bytes=100 * 1024 * 1024),
    )
    s64, attn = iters_call(
        params['slot_inits'], k3, v3,
        params['Wq'].astype(BF),
        row(params['slot_ln_g']), row(params['slot_ln_b']),
        params['W_ih'].astype(BF), params['W_hh'].astype(BF),
        row(params['b_ih']), row(params['b_hh']),
        row(params['mlp_ln_g']), row(params['mlp_ln_b']),
        params['mlp_W1'].astype(BF), row(params['mlp_b1']),
        params['mlp_W2'].astype(BF), row(params['mlp_b2']))
    slots = s64.reshape(B, K, D)

    sel_call = pl.pallas_call(
        functools.partial(_sel_quality_kernel, B=B, K=K),
        in_specs=[
            full((B * K, D)), full((B, K, N)),
            full((D // 2, D)), full((1, D // 2)),
            full((1, D // 2)), full((1, 1)),
            full((D // 4, D)), full((1, D // 4)),
            full((1, D // 4)), full((1, 1)),
        ],
        out_specs=[full((B * K, 1)), full((B * K, 1))],
        out_shape=[
            jax.ShapeDtypeStruct((B * K, 1), f32),
            jax.ShapeDtypeStruct((B * K, 1), f32),
        ],
    )
    mask64, qual64 = sel_call(
        s64, attn,
        params['sel_W1'], row(params['sel_b1']),
        params['sel_W2'], row(params['sel_b2']),
        params['q_W1'], row(params['q_b1']),
        params['q_W2'], row(params['q_b2']))

    return slots, mask64.reshape(B, K), qual64.reshape(B, K), attn


# speedup vs baseline: 4.4852x; 1.1834x over previous
import functools

import jax
import jax.numpy as jnp
from jax.experimental import pallas as pl
from jax.experimental.pallas import tpu as pltpu

HI = jax.lax.Precision.HIGHEST
BF = jnp.bfloat16


def _dot_t(a, b, precision=HI):
    return jax.lax.dot_general(a, b, (((1,), (1,)), ((), ())),
                               precision=precision,
                               preferred_element_type=jnp.float32)


def _bdot_t(a, b):
    return jax.lax.dot_general(a.astype(BF), b, (((1,), (1,)), ((), ())),
                               preferred_element_type=jnp.float32)


def _layernorm(x, g, b, eps=1e-5):
    m = x.mean(-1, keepdims=True)
    v = ((x - m) ** 2).mean(-1, keepdims=True)
    return (x - m) * jax.lax.rsqrt(v + eps) * g + b


def _kv_kernel(x_ref, g_ref, b_ref, wk_ref, wv_ref,
               wih_ref, whh_ref, w1_ref, w2_ref, wq_ref,
               k_ref, v_ref, wihb_ref, whhb_ref, w1b_ref, w2b_ref, wqb_ref,
               wk_bf, wv_bf, *, half):
    @pl.when(pl.program_id(0) == 0)
    def _():
        wk_bf[...] = wk_ref[...].astype(BF)
        wv_bf[...] = wv_ref[...].astype(BF)

    wihb_ref[...] = wih_ref[...].astype(BF)
    whhb_ref[...] = whh_ref[...].astype(BF)
    w1b_ref[...] = w1_ref[...].astype(BF)
    w2b_ref[...] = w2_ref[...].astype(BF)
    wqb_ref[...] = wq_ref[...].astype(BF)

    g = g_ref[...]
    b = b_ref[...]
    xn0 = _layernorm(x_ref[:half, :], g, b).astype(BF)
    k_ref[:half, :] = _bdot_t(xn0, wk_bf[...]).astype(BF)
    v_ref[:half, :] = _bdot_t(xn0, wv_bf[...]).astype(BF)
    xn1 = _layernorm(x_ref[half:, :], g, b).astype(BF)
    k_ref[half:, :] = _bdot_t(xn1, wk_bf[...]).astype(BF)
    v_ref[half:, :] = _bdot_t(xn1, wv_bf[...]).astype(BF)


def _iters_kernel(si_ref, k_ref, v_ref, wq_ref, sg_ref, sb_ref,
                  wih_ref, whh_ref, bih_ref, bhh_ref,
                  mg_ref, mb_ref, w1_ref, b1_ref, w2_ref, b2_ref,
                  slots_ref, attn_ref, *, B, K, N, D, scale, iters):
    si = si_ref[...]
    h = jnp.concatenate([si] * B, axis=0)
    for it in range(iters):
        sn = _layernorm(h, sg_ref[...], sb_ref[...])
        q = _bdot_t(sn, wq_ref[...])
        upds = []
        for b in range(B):
            logits = jax.lax.dot_general(
                q[b * K:(b + 1) * K, :].astype(BF), k_ref[b],
                (((1,), (1,)), ((), ())),
                preferred_element_type=jnp.float32) * scale
            m = jnp.max(logits, axis=0, keepdims=True)
            e = jnp.exp(logits - m)
            attn_b = e / jnp.sum(e, axis=0, keepdims=True)
            attn_ref[b] = attn_b
            w = attn_b / (jnp.sum(attn_b, axis=1, keepdims=True) + 1e-8)
            upds.append(jax.lax.dot_general(
                w.astype(BF), v_ref[b], (((1,), (0,)), ((), ())),
                preferred_element_type=jnp.float32))
        u = jnp.concatenate(upds, axis=0)
        gi = _bdot_t(u, wih_ref[...]) + bih_ref[...]
        gh = _bdot_t(h, whh_ref[...]) + bhh_ref[...]
        r = jax.nn.sigmoid(gi[:, :D] + gh[:, :D])
        z = jax.nn.sigmoid(gi[:, D:2 * D] + gh[:, D:2 * D])
        n = jnp.tanh(gi[:, 2 * D:] + r * gh[:, 2 * D:])
        s = (1.0 - z) * n + z * h
        hm = _layernorm(s, mg_ref[...], mb_ref[...])
        h1 = jnp.maximum(_bdot_t(hm, w1_ref[...]) + b1_ref[...], 0.0)
        h = s + _bdot_t(h1, w2_ref[...]) + b2_ref[...]
    slots_ref[...] = h


def _sel_quality_kernel(slots_ref, attn_ref, sw1_ref, sb1_ref, sw2_ref,
                        sb2_ref, qw1_ref, qb1_ref, qw2_ref, qb2_ref,
                        mask_ref, qual_ref, *, B, K):
    h = slots_ref[...]
    sh = jnp.maximum(_dot_t(h, sw1_ref[...]) + sb1_ref[...], 0.0)
    sel_logits = (jnp.sum(sh * sw2_ref[...], axis=1, keepdims=True)
                  + sb2_ref[0, 0])
    sel_probs = jax.nn.sigmoid(sel_logits)
    mask = jnp.where(sel_probs > 0.5, 1.0, 0.0)

    qh = jnp.maximum(_dot_t(h, qw1_ref[...]) + qb1_ref[...], 0.0)
    learned_q = jax.nn.sigmoid(
        jnp.sum(qh * qw2_ref[...], axis=1, keepdims=True) + qb2_ref[0, 0])

    row_iota = jax.lax.broadcasted_iota(jnp.int32, (K, 1), 0)
    min_sum = None
    fbs = []
    for b in range(B):
        lo = b * K
        lg = sel_logits[lo:lo + K, :]
        msum = jnp.sum(mask[lo:lo + K, :])
        min_sum = msum if min_sum is None else jnp.minimum(min_sum, msum)
        mx = jnp.max(lg)
        idx = jnp.min(jnp.where(lg == mx, row_iota, K))
        fbs.append(jnp.where(row_iota == idx, 1.0, 0.0))
        a = attn_ref[b]
        amax = jnp.max(a, axis=0, keepdims=True)
        kio = jax.lax.broadcasted_iota(jnp.int32, a.shape, 0)
        widx = jnp.min(jnp.where(a == amax, kio, K),
                       axis=0, keepdims=True)
        onehot = jnp.where(kio == widx, 1.0, 0.0)
        winning = jnp.sum(a * onehot, axis=1, keepdims=True)
        total = jnp.sum(a, axis=1, keepdims=True)
        aq = winning / (total + 1e-8)
        qual_ref[lo:lo + K, :] = 0.4 * aq + 0.4 * learned_q[lo:lo + K, :]

    need_fb = min_sum < 1.0
    for b in range(B):
        lo = b * K
        mask_ref[lo:lo + K, :] = jnp.where(need_fb, fbs[b], mask[lo:lo + K, :])


def kernel(inputs, params):
    B, N, D = inputs.shape
    K = params['slot_inits'].shape[0]
    scale = D ** -0.5
    f32 = jnp.float32

    def row(p):
        return p.reshape(1, -1)

    TM = 512
    G = B * N // TM
    x2 = inputs.reshape(B * N, D)
    cst = lambda shape: pl.BlockSpec(shape, lambda i: tuple(0 for _ in shape))
    chunk = lambda rows, cols: pl.BlockSpec((rows, cols), lambda i: (i, 0))
    kv = pl.pallas_call(
        functools.partial(_kv_kernel, half=TM // 2),
        grid=(G,),
        in_specs=[
            chunk(TM, D),
            cst((1, D)), cst((1, D)),
            cst((D, D)), cst((D, D)),
            chunk(3 * D // G, D),
            chunk(3 * D // G, D),
            chunk(2 * D // G, D),
            chunk(D // G, 2 * D),
            chunk(D // G, D),
        ],
        out_specs=[
            chunk(TM, D), chunk(TM, D),
            chunk(3 * D // G, D), chunk(3 * D // G, D),
            chunk(2 * D // G, D), chunk(D // G, 2 * D), chunk(D // G, D),
        ],
        out_shape=[
            jax.ShapeDtypeStruct((B * N, D), BF),
            jax.ShapeDtypeStruct((B * N, D), BF),
            jax.ShapeDtypeStruct((3 * D, D), BF),
            jax.ShapeDtypeStruct((3 * D, D), BF),
            jax.ShapeDtypeStruct((2 * D, D), BF),
            jax.ShapeDtypeStruct((D, 2 * D), BF),
            jax.ShapeDtypeStruct((D, D), BF),
        ],
        scratch_shapes=[pltpu.VMEM((D, D), BF), pltpu.VMEM((D, D), BF)],
    )
    k2, v2, wih_b, whh_b, w1_b, w2_b, wq_b = kv(
        x2, row(params['in_ln_g']), row(params['in_ln_b']),
        params['Wk'], params['Wv'],
        params['W_ih'], params['W_hh'],
        params['mlp_W1'], params['mlp_W2'], params['Wq'])
    k3 = k2.reshape(B, N, D)
    v3 = v2.reshape(B, N, D)

    full = lambda shape: pl.BlockSpec(shape, lambda: tuple(0 for _ in shape))
    iters_call = pl.pallas_call(
        functools.partial(_iters_kernel, B=B, K=K, N=N, D=D,
                          scale=scale, iters=3),
        in_specs=[
            full((K, D)), full((B, N, D)), full((B, N, D)),
            full((D, D)), full((1, D)), full((1, D)),
            full((3 * D, D)), full((3 * D, D)),
            full((1, 3 * D)), full((1, 3 * D)),
            full((1, D)), full((1, D)),
            full((2 * D, D)), full((1, 2 * D)),
            full((D, 2 * D)), full((1, D)),
        ],
        out_specs=[full((B * K, D)), full((B, K, N))],
        out_shape=[
            jax.ShapeDtypeStruct((B * K, D), f32),
            jax.ShapeDtypeStruct((B, K, N), f32),
        ],
        compiler_params=pltpu.CompilerParams(
            vmem_limit_bytes=100 * 1024 * 1024),
    )
    s64, attn = iters_call(
        params['slot_inits'], k3, v3, wq_b,
        row(params['slot_ln_g']), row(params['slot_ln_b']),
        wih_b, whh_b, row(params['b_ih']), row(params['b_hh']),
        row(params['mlp_ln_g']), row(params['mlp_ln_b']),
        w1_b, row(params['mlp_b1']), w2_b, row(params['mlp_b2']))

    sel_call = pl.pallas_call(
        functools.partial(_sel_quality_kernel, B=B, K=K),
        in_specs=[
            full((B * K, D)), full((B, K, N)),
            full((D // 2, D)), full((1, D // 2)),
            full((1, D // 2)), full((1, 1)),
            full((D // 4, D)), full((1, D // 4)),
            full((1, D // 4)), full((1, 1)),
        ],
        out_specs=[full((B * K, 1)), full((B * K, 1))],
        out_shape=[
            jax.ShapeDtypeStruct((B * K, 1), f32),
            jax.ShapeDtypeStruct((B * K, 1), f32),
        ],
    )
    mask64, qual64 = sel_call(
        s64, attn,
        params['sel_W1'], row(params['sel_b1']),
        params['sel_W2'], row(params['sel_b2']),
        params['q_W1'], row(params['q_b1']),
        params['q_W2'], row(params['q_b2']))

    return (s64.reshape(B, K, D), mask64.reshape(B, K),
            qual64.reshape(B, K), attn)
